# baseline (speedup 1.0000x reference)
import functools

import jax
import jax.numpy as jnp
from jax import lax
from jax.experimental import pallas as pl
from jax.experimental.pallas import tpu as pltpu
from jax.experimental.pallas import tpu_sc as plsc

VOCAB = 1000000
EMB_DIM = 64
N_GARMENT = 21
N_INDEX = 5
BATCH = 16384

NC, NS = 2, 16
NW = NC * NS
BPW = BATCH // NW
CHUNK = 128
NCHUNK = BPW // CHUNK

G_PAD = 32
I_PAD = 8

_sc_mesh = plsc.VectorSubcoreMesh(core_axis_name="c", subcore_axis_name="s")


@functools.partial(
    pl.kernel,
    out_type=jax.ShapeDtypeStruct((NW, NCHUNK, CHUNK, EMB_DIM), jnp.float32),
    mesh=_sc_mesh,
    scratch_types=[
        pltpu.VMEM((NCHUNK, CHUNK), jnp.int32),
        pltpu.VMEM((NCHUNK, CHUNK, EMB_DIM), jnp.float32),
        pltpu.SemaphoreType.DMA,
    ],
    compiler_params=pltpu.CompilerParams(use_tc_tiling_on_sc=False),
)
def _sc_gather(table_hbm, idx_hbm, out_hbm, idx_v, rows_v, sem):
    wid = lax.axis_index("s") * NC + lax.axis_index("c")
    pltpu.sync_copy(idx_hbm.at[wid], idx_v)
    copies = []
    for j in range(NCHUNK):
        copies.append(
            pltpu.async_copy(table_hbm.at[idx_v.at[j]], rows_v.at[j], sem)
        )
    for c in copies:
        c.wait()
    pltpu.sync_copy(rows_v, out_hbm.at[wid])


def _tc_mlp_body(x_ref, g_ref, i_ref, w1a_ref, w1g_ref, w1i_ref, b1_ref,
                 w2_ref, b2_ref, o_ref):
    x = x_ref[...]
    gid = g_ref[...]
    iid = i_ref[...]
    blk = x.shape[0]
    goh = (gid == lax.broadcasted_iota(jnp.int32, (blk, G_PAD), 1)
           ).astype(jnp.float32)
    ioh = (iid == lax.broadcasted_iota(jnp.int32, (blk, I_PAD), 1)
           ).astype(jnp.float32)
    h = jnp.dot(x, w1a_ref[...], preferred_element_type=jnp.float32)
    h += jnp.dot(goh, w1g_ref[...], preferred_element_type=jnp.float32)
    h += jnp.dot(ioh, w1i_ref[...], preferred_element_type=jnp.float32)
    h = jnp.maximum(h + b1_ref[...], 0.0)
    o_ref[...] = jnp.dot(h, w2_ref[...],
                         preferred_element_type=jnp.float32) + b2_ref[...]


def kernel(article_id, garment_group_name, index_group_name, emb_table,
           W1, b1, W2, b2):
    idx = article_id.astype(jnp.int32).reshape(NW, NCHUNK, CHUNK)
    gathered = _sc_gather(emb_table, idx).reshape(BATCH, EMB_DIM)

    w1a = W1[:EMB_DIM]
    w1g = jnp.zeros((G_PAD, EMB_DIM), jnp.float32).at[:N_GARMENT].set(
        W1[EMB_DIM:EMB_DIM + N_GARMENT])
    w1i = jnp.zeros((I_PAD, EMB_DIM), jnp.float32).at[:N_INDEX].set(
        W1[EMB_DIM + N_GARMENT:])
    gid = garment_group_name.astype(jnp.int32).reshape(BATCH, 1)
    iid = index_group_name.astype(jnp.int32).reshape(BATCH, 1)

    BLK = 2048
    grid = (BATCH // BLK,)
    out = pl.pallas_call(
        _tc_mlp_body,
        grid=grid,
        in_specs=[
            pl.BlockSpec((BLK, EMB_DIM), lambda i: (i, 0)),
            pl.BlockSpec((BLK, 1), lambda i: (i, 0)),
            pl.BlockSpec((BLK, 1), lambda i: (i, 0)),
            pl.BlockSpec((EMB_DIM, EMB_DIM), lambda i: (0, 0)),
            pl.BlockSpec((G_PAD, EMB_DIM), lambda i: (0, 0)),
            pl.BlockSpec((I_PAD, EMB_DIM), lambda i: (0, 0)),
            pl.BlockSpec((1, EMB_DIM), lambda i: (0, 0)),
            pl.BlockSpec((EMB_DIM, EMB_DIM), lambda i: (0, 0)),
            pl.BlockSpec((1, EMB_DIM), lambda i: (0, 0)),
        ],
        out_specs=pl.BlockSpec((BLK, EMB_DIM), lambda i: (i, 0)),
        out_shape=jax.ShapeDtypeStruct((BATCH, EMB_DIM), jnp.float32),
    )(gathered, gid, iid, w1a, w1g, w1i, b1.reshape(1, EMB_DIM), W2,
      b2.reshape(1, EMB_DIM))
    return out
